# baseline (speedup 1.0000x reference)
import jax
import jax.numpy as jnp
from jax import lax
from jax.experimental import pallas as pl
from jax.experimental.pallas import tpu as pltpu
from jax.experimental.pallas import tpu_sc as plsc

_NC = 2
_NS = 16
_NW = _NC * _NS
_L = 16
_B = 16384
_D = 32
_BPW = _B // _NW
_CH = 128
_NCHUNK = _BPW // _CH
_NGROUP = _BPW // _L


def _gmf_body(users, movies, utab, mtab, w, b, out,
              uidx, midx, urows, mrows, wv, bv, outv, tbuf, sem):
    wid = lax.axis_index("s") * _NC + lax.axis_index("c")
    base = wid * _BPW

    pltpu.sync_copy(w, wv)
    pltpu.sync_copy(b, bv)
    for c in range(_NCHUNK):
        pltpu.sync_copy(users.at[pl.ds(base + c * _CH, _CH)], uidx.at[c])
        pltpu.sync_copy(movies.at[pl.ds(base + c * _CH, _CH)], midx.at[c])

    copies = []
    for c in range(_NCHUNK):
        copies.append(pltpu.async_copy(
            utab.at[uidx.at[c]], urows.at[pl.ds(c * _CH, _CH)], sem))
        copies.append(pltpu.async_copy(
            mtab.at[midx.at[c]], mrows.at[pl.ds(c * _CH, _CH)], sem))
    for cp in copies:
        cp.wait()

    w0 = wv[pl.ds(0, _L)]
    w1 = wv[pl.ds(_L, _L)]
    bias = bv[...]
    iota16 = lax.iota(jnp.int32, _L) * _L

    def group_body(g, carry):
        b0 = g * _L
        for j in range(_L):
            r = b0 + j
            u0 = urows[r, pl.ds(0, _L)]
            u1 = urows[r, pl.ds(_L, _L)]
            m0 = mrows[r, pl.ds(0, _L)]
            m1 = mrows[r, pl.ds(_L, _L)]
            s = u0 * m0 * w0 + u1 * m1 * w1
            plsc.store_scatter(tbuf, [iota16 + j], s)
        acc = tbuf[pl.ds(0, _L)]
        for lane in range(1, _L):
            acc = acc + tbuf[pl.ds(lane * _L, _L)]
        x = acc + bias
        y = 1.0 / (1.0 + jnp.exp(-x))
        outv[pl.ds(b0, _L)] = y
        return carry

    lax.fori_loop(0, _NGROUP, group_body, 0)
    pltpu.sync_copy(outv, out.at[pl.ds(base, _BPW)])


def kernel(users, movies, user_table, movie_table, lin_w, lin_b):
    mesh = plsc.VectorSubcoreMesh(core_axis_name="c", subcore_axis_name="s")
    f = pl.kernel(
        _gmf_body,
        mesh=mesh,
        compiler_params=pltpu.CompilerParams(
            needs_layout_passes=False, use_tc_tiling_on_sc=False),
        out_type=jax.ShapeDtypeStruct((_B,), jnp.float32),
        scratch_types=[
            pltpu.VMEM((_NCHUNK, _CH), jnp.int32),
            pltpu.VMEM((_NCHUNK, _CH), jnp.int32),
            pltpu.VMEM((_BPW, _D), jnp.float32),
            pltpu.VMEM((_BPW, _D), jnp.float32),
            pltpu.VMEM((_D,), jnp.float32),
            pltpu.VMEM((_L,), jnp.float32),
            pltpu.VMEM((_BPW,), jnp.float32),
            pltpu.VMEM((_L * _L,), jnp.float32),
            pltpu.SemaphoreType.DMA,
        ],
    )
    out = f(users.astype(jnp.int32), movies.astype(jnp.int32),
            user_table, movie_table,
            lin_w.reshape(_D).astype(jnp.float32),
            jnp.broadcast_to(lin_b.astype(jnp.float32).reshape(()), (_L,)))
    return out.reshape(_B, 1)
